# Initial kernel scaffold; baseline (speedup 1.0000x reference)
import jax
import jax.numpy as jnp
from jax.experimental import pallas as pl


def kernel(x, emb, W_ih, W_hh, b_ih, b_hh, W_fc, b_fc):
    raise NotImplementedError("write your pallas kernel here")


# speedup vs baseline: 3.0055x; 3.0055x over previous
import jax
import jax.numpy as jnp
from jax.experimental import pallas as pl
from jax.experimental.pallas import tpu as pltpu

VOCAB = 256
EMBED = 256
HIDDEN = 512
SEQ = 256
BATCH = 64


def _lstm_kernel(xs_ref, emb_ref, wih_ref, whh_ref, bih_ref, bhh_ref,
                 wfc_ref, bfc_ref, out_ref, h_ref, c_ref, table_ref):
    t = pl.program_id(0)

    @pl.when(t == 0)
    def _init():
        bias = bih_ref[...] + bhh_ref[...]
        table_ref[...] = jnp.dot(emb_ref[...], wih_ref[...],
                                 preferred_element_type=jnp.float32) + bias
        h_ref[...] = jnp.zeros_like(h_ref)
        c_ref[...] = jnp.zeros_like(c_ref)

    xt = xs_ref[0, 0, :]
    onehot = (xt[:, None] == jax.lax.broadcasted_iota(
        jnp.int32, (BATCH, VOCAB), 1)).astype(jnp.float32)
    inp = jnp.dot(onehot, table_ref[...], preferred_element_type=jnp.float32)
    gates = inp + jnp.dot(h_ref[...], whh_ref[...],
                          preferred_element_type=jnp.float32)
    i = jax.nn.sigmoid(gates[:, 0 * HIDDEN:1 * HIDDEN])
    f = jax.nn.sigmoid(gates[:, 1 * HIDDEN:2 * HIDDEN])
    g = jnp.tanh(gates[:, 2 * HIDDEN:3 * HIDDEN])
    o = jax.nn.sigmoid(gates[:, 3 * HIDDEN:4 * HIDDEN])
    c_new = f * c_ref[...] + i * g
    h_new = o * jnp.tanh(c_new)
    c_ref[...] = c_new
    h_ref[...] = h_new

    @pl.when(t == SEQ - 1)
    def _fin():
        out_ref[...] = jnp.dot(h_new, wfc_ref[...],
                               preferred_element_type=jnp.float32) + bfc_ref[...]


def kernel(x, emb, W_ih, W_hh, b_ih, b_hh, W_fc, b_fc):
    xs = jnp.transpose(x.astype(jnp.int32), (1, 0)).reshape(SEQ, 1, BATCH)
    wih_t = W_ih.T
    whh_t = W_hh.T
    wfc_t = W_fc.T
    bih = b_ih.reshape(1, 4 * HIDDEN)
    bhh = b_hh.reshape(1, 4 * HIDDEN)
    bfc = b_fc.reshape(1, VOCAB)

    grid = (SEQ,)
    in_specs = [
            pl.BlockSpec((1, 1, BATCH), lambda t: (t, 0, 0)),
            pl.BlockSpec((VOCAB, EMBED), lambda t: (0, 0)),
            pl.BlockSpec((EMBED, 4 * HIDDEN), lambda t: (0, 0)),
            pl.BlockSpec((HIDDEN, 4 * HIDDEN), lambda t: (0, 0)),
            pl.BlockSpec((1, 4 * HIDDEN), lambda t: (0, 0)),
            pl.BlockSpec((1, 4 * HIDDEN), lambda t: (0, 0)),
            pl.BlockSpec((HIDDEN, VOCAB), lambda t: (0, 0)),
            pl.BlockSpec((1, VOCAB), lambda t: (0, 0)),
    ]

    return pl.pallas_call(
        _lstm_kernel,
        grid=grid,
        in_specs=in_specs,
        out_specs=pl.BlockSpec((BATCH, VOCAB), lambda t: (0, 0)),
        out_shape=jax.ShapeDtypeStruct((BATCH, VOCAB), jnp.float32),
        scratch_shapes=[
            pltpu.VMEM((BATCH, HIDDEN), jnp.float32),
            pltpu.VMEM((BATCH, HIDDEN), jnp.float32),
            pltpu.VMEM((VOCAB, 4 * HIDDEN), jnp.float32),
        ],
    )(xs, emb, wih_t, whh_t, bih, bhh, wfc_t, bfc)


# speedup vs baseline: 3.1186x; 1.0376x over previous
import jax
import jax.numpy as jnp
from jax.experimental import pallas as pl
from jax.experimental.pallas import tpu as pltpu

VOCAB = 256
EMBED = 256
HIDDEN = 512
SEQ = 256
BATCH = 64


def _lstm_kernel(xs_ref, emb_ref, wih_ref, whh_ref, bih_ref, bhh_ref,
                 wfc_ref, bfc_ref, out_ref, table_ref):
    bias = bih_ref[...] + bhh_ref[...]
    table_ref[...] = jnp.dot(emb_ref[...], wih_ref[...],
                             preferred_element_type=jnp.float32) + bias

    def step(t, carry):
        h, c = carry
        xt = xs_ref[t, 0, :]
        onehot = (xt[:, None] == jax.lax.broadcasted_iota(
            jnp.int32, (BATCH, VOCAB), 1)).astype(jnp.float32)
        inp = jnp.dot(onehot, table_ref[...],
                      preferred_element_type=jnp.float32)
        gates = inp + jnp.dot(h, whh_ref[...],
                              preferred_element_type=jnp.float32)
        i = jax.nn.sigmoid(gates[:, 0 * HIDDEN:1 * HIDDEN])
        f = jax.nn.sigmoid(gates[:, 1 * HIDDEN:2 * HIDDEN])
        g = jnp.tanh(gates[:, 2 * HIDDEN:3 * HIDDEN])
        o = jax.nn.sigmoid(gates[:, 3 * HIDDEN:4 * HIDDEN])
        c_new = f * c + i * g
        h_new = o * jnp.tanh(c_new)
        return h_new, c_new

    h0 = jnp.zeros((BATCH, HIDDEN), dtype=jnp.float32)
    c0 = jnp.zeros((BATCH, HIDDEN), dtype=jnp.float32)
    h_last, _ = jax.lax.fori_loop(0, SEQ, step, (h0, c0))
    out_ref[...] = jnp.dot(h_last, wfc_ref[...],
                           preferred_element_type=jnp.float32) + bfc_ref[...]


def kernel(x, emb, W_ih, W_hh, b_ih, b_hh, W_fc, b_fc):
    xs = jnp.transpose(x.astype(jnp.int32), (1, 0)).reshape(SEQ, 1, BATCH)
    wih_t = W_ih.T
    whh_t = W_hh.T
    wfc_t = W_fc.T
    bih = b_ih.reshape(1, 4 * HIDDEN)
    bhh = b_hh.reshape(1, 4 * HIDDEN)
    bfc = b_fc.reshape(1, VOCAB)

    return pl.pallas_call(
        _lstm_kernel,
        out_shape=jax.ShapeDtypeStruct((BATCH, VOCAB), jnp.float32),
        scratch_shapes=[
            pltpu.VMEM((VOCAB, 4 * HIDDEN), jnp.float32),
        ],
    )(xs, emb, wih_t, whh_t, bih, bhh, wfc_t, bfc)


# speedup vs baseline: 3.1442x; 1.0082x over previous
import jax
import jax.numpy as jnp
from jax.experimental import pallas as pl
from jax.experimental.pallas import tpu as pltpu

VOCAB = 256
EMBED = 256
HIDDEN = 512
SEQ = 256
BATCH = 64


def _lstm_kernel(xs_ref, emb_ref, wih_ref, whh_ref, bih_ref, bhh_ref,
                 wfc_ref, bfc_ref, out_ref, table_ref):
    bias = bih_ref[...] + bhh_ref[...]
    table_f32 = jnp.dot(emb_ref[...], wih_ref[...],
                        preferred_element_type=jnp.float32) + bias
    table_ref[...] = table_f32.astype(jnp.bfloat16)

    def step(t, carry):
        h, c = carry
        xt = xs_ref[t, 0, :]
        onehot = (xt[:, None] == jax.lax.broadcasted_iota(
            jnp.int32, (BATCH, VOCAB), 1)).astype(jnp.bfloat16)
        inp = jnp.dot(onehot, table_ref[...],
                      preferred_element_type=jnp.float32)
        gates = inp + jnp.dot(h.astype(jnp.bfloat16), whh_ref[...],
                              preferred_element_type=jnp.float32)
        i = jax.nn.sigmoid(gates[:, 0 * HIDDEN:1 * HIDDEN])
        f = jax.nn.sigmoid(gates[:, 1 * HIDDEN:2 * HIDDEN])
        g = jnp.tanh(gates[:, 2 * HIDDEN:3 * HIDDEN])
        o = jax.nn.sigmoid(gates[:, 3 * HIDDEN:4 * HIDDEN])
        c_new = f * c + i * g
        h_new = o * jnp.tanh(c_new)
        return h_new, c_new

    h0 = jnp.zeros((BATCH, HIDDEN), dtype=jnp.float32)
    c0 = jnp.zeros((BATCH, HIDDEN), dtype=jnp.float32)
    h_last, _ = jax.lax.fori_loop(0, SEQ, step, (h0, c0))
    out_ref[...] = jnp.dot(h_last, wfc_ref[...],
                           preferred_element_type=jnp.float32) + bfc_ref[...]


def kernel(x, emb, W_ih, W_hh, b_ih, b_hh, W_fc, b_fc):
    xs = jnp.transpose(x.astype(jnp.int32), (1, 0)).reshape(SEQ, 1, BATCH)
    wih_t = W_ih.T
    whh_t = W_hh.T.astype(jnp.bfloat16)
    wfc_t = W_fc.T
    bih = b_ih.reshape(1, 4 * HIDDEN)
    bhh = b_hh.reshape(1, 4 * HIDDEN)
    bfc = b_fc.reshape(1, VOCAB)

    return pl.pallas_call(
        _lstm_kernel,
        out_shape=jax.ShapeDtypeStruct((BATCH, VOCAB), jnp.float32),
        scratch_shapes=[
            pltpu.VMEM((VOCAB, 4 * HIDDEN), jnp.bfloat16),
        ],
    )(xs, emb, wih_t, whh_t, bih, bhh, wfc_t, bfc)
